# baseline (speedup 1.0000x reference)
import functools

import jax
import jax.numpy as jnp
from jax import lax
from jax.experimental import pallas as pl
from jax.experimental.pallas import tpu as pltpu
from jax.experimental.pallas import tpu_sc as plsc

N = 10000
E = 320000
D_IN = 128
HID = 64
NG = 64
GF = 128
N_PAD = 10240
NC = 2
NS = 16
NW = NC * NS
CH = 400
ROWS_T = 25
EPT = ROWS_T * CH
E_PAD = NW * EPT
PADN = N_PAD - 1
RPT = N_PAD // NS

def _deg_body(dst_hbm, ones_hbm, out_hbm, dst_all, ones_v, bounce, acc):
    c = lax.axis_index("c")
    s = lax.axis_index("s")
    wid = s * NC + c
    pltpu.sync_copy(dst_hbm.at[wid], dst_all)
    pltpu.sync_copy(ones_hbm.at[pl.ds(0, CH)], ones_v)
    pltpu.sync_copy(ones_hbm.at[pl.ds(s * RPT, RPT)], bounce)
    pltpu.sync_copy(bounce, acc.at[pl.ds(s * RPT, RPT)])
    plsc.subcore_barrier()

    def chunk(i, carry):
        pltpu.sync_copy(ones_v, acc.at[dst_all.at[i]], add=True)
        return carry

    lax.fori_loop(0, ROWS_T, chunk, 0)
    plsc.subcore_barrier()
    pltpu.sync_copy(acc.at[pl.ds(s * RPT, RPT)], bounce)
    pltpu.sync_copy(bounce, out_hbm.at[c].at[pl.ds(s * RPT, RPT)])


def _scatter_body(p_hbm, src_hbm, dst_hbm, out_hbm,
                  src_all, dst_all, rows0, rows1, bounce, acc,
                  sem0, sem1, sem2, sem3):
    c = lax.axis_index("c")
    s = lax.axis_index("s")
    wid = s * NC + c
    pltpu.sync_copy(src_hbm.at[wid], src_all)
    pltpu.sync_copy(dst_hbm.at[wid], dst_all)
    pltpu.sync_copy(p_hbm.at[pl.ds(s * RPT, RPT)], bounce)
    pltpu.sync_copy(bounce, acc.at[pl.ds(s * RPT, RPT)])
    plsc.subcore_barrier()

    def g_start(j, buf, sem):
        pltpu.async_copy(p_hbm.at[src_all.at[j]], buf, sem)

    def g_wait(buf, sem):
        pltpu.make_async_copy(p_hbm.at[src_all.at[0]], buf, sem).wait()

    g_start(0, rows0, sem0)

    def pair(k, carry):
        i = 2 * k
        g_start(i + 1, rows1, sem1)
        g_wait(rows0, sem0)
        pltpu.sync_copy(rows0, acc.at[dst_all.at[i]], add=True)
        g_start(i + 2, rows0, sem0)
        g_wait(rows1, sem1)
        pltpu.sync_copy(rows1, acc.at[dst_all.at[i + 1]], add=True)
        return carry

    lax.fori_loop(0, (ROWS_T - 1) // 2, pair, 0)
    g_wait(rows0, sem0)
    pltpu.sync_copy(rows0, acc.at[dst_all.at[ROWS_T - 1]], add=True)
    plsc.subcore_barrier()
    pltpu.sync_copy(acc.at[pl.ds(s * RPT, RPT)], bounce)
    pltpu.sync_copy(bounce, out_hbm.at[c].at[pl.ds(s * RPT, RPT)])


@functools.cache
def _sc_kernels():
    mesh = plsc.VectorSubcoreMesh(core_axis_name="c", subcore_axis_name="s")
    deg_kernel = pl.kernel(
        _deg_body,
        mesh=mesh,
        compiler_params=pltpu.CompilerParams(use_tc_tiling_on_sc=False),
        out_type=jax.ShapeDtypeStruct((NC, N_PAD, 8), jnp.float32),
        scratch_types=[
            pltpu.VMEM((ROWS_T, CH), jnp.int32),
            pltpu.VMEM((CH, 8), jnp.float32),
            pltpu.VMEM((RPT, 8), jnp.float32),
            pltpu.VMEM_SHARED((N_PAD, 8), jnp.float32),
        ],
    )
    scatter_kernel = pl.kernel(
        _scatter_body,
        mesh=mesh,
        compiler_params=pltpu.CompilerParams(use_tc_tiling_on_sc=False),
        out_type=jax.ShapeDtypeStruct((NC, N_PAD, HID), jnp.bfloat16),
        scratch_types=[
            pltpu.VMEM((ROWS_T, CH), jnp.int32),
            pltpu.VMEM((ROWS_T, CH), jnp.int32),
            pltpu.VMEM((CH, HID), jnp.bfloat16),
            pltpu.VMEM((CH, HID), jnp.bfloat16),
            pltpu.VMEM((RPT, HID), jnp.bfloat16),
            pltpu.VMEM_SHARED((N_PAD, HID), jnp.bfloat16),
            pltpu.SemaphoreType.DMA,
            pltpu.SemaphoreType.DMA,
            pltpu.SemaphoreType.DMA,
            pltpu.SemaphoreType.DMA,
        ],
    )
    return deg_kernel, scatter_kernel


def _stage_a(x_ref, w1_ref, h_ref):
    h_ref[...] = jnp.dot(x_ref[...], w1_ref[...],
                         preferred_element_type=jnp.float32)


def _stage_b(h_ref, deg_ref, p_ref, dinv_ref):
    deg = deg_ref[0, :, 0:1] + deg_ref[1, :, 0:1] - 1.0
    dinv = lax.rsqrt(deg)
    p_ref[...] = (h_ref[...] * dinv).astype(jnp.bfloat16)
    dinv_ref[...] = dinv


def _stage_d(s_ref, p_ref, dinv_ref, b1_ref, w2_ref, p2_ref):
    dinv = dinv_ref[...]
    edge_sum = (s_ref[0].astype(jnp.float32) + s_ref[1].astype(jnp.float32)
                - p_ref[...].astype(jnp.float32))
    z = jnp.maximum(edge_sum * dinv + b1_ref[...], 0.0)
    h2 = jnp.dot(z, w2_ref[...], preferred_element_type=jnp.float32)
    p2_ref[...] = (h2 * dinv).astype(jnp.bfloat16)


def _stage_f(s_ref, p_ref, dinv_ref, b2_ref, batch_ref, gf_ref, wg_ref,
             bg_ref, gamma_ref, beta_ref, wc1_ref, bc1_ref, wc2_ref,
             bc2_ref, out_ref):
    edge_sum = (s_ref[0].astype(jnp.float32) + s_ref[1].astype(jnp.float32)
                - p_ref[...].astype(jnp.float32))
    z2 = jnp.maximum(edge_sum * dinv_ref[...] + b2_ref[...], 0.0)
    seg = lax.broadcasted_iota(jnp.int32, (NG, N_PAD), 0)
    oh = (seg == batch_ref[...]).astype(jnp.float32)
    sums = jnp.dot(oh, z2, preferred_element_type=jnp.float32)
    cnt = jnp.sum(oh, axis=1, keepdims=True)
    node_feat = sums / jnp.maximum(cnt, 1.0)
    g = jnp.dot(gf_ref[...], wg_ref[...],
                preferred_element_type=jnp.float32) + bg_ref[...]
    mean = jnp.mean(g, axis=0, keepdims=True)
    var = jnp.mean((g - mean) * (g - mean), axis=0, keepdims=True)
    g = gamma_ref[...] * (g - mean) / jnp.sqrt(var + 1e-5) + beta_ref[...]
    g = jnp.maximum(g, 0.0)
    comb = jnp.concatenate([node_feat, g], axis=1)
    hidden = jnp.maximum(
        jnp.dot(comb, wc1_ref[...], preferred_element_type=jnp.float32)
        + bc1_ref[...], 0.0)
    out_ref[...] = (jnp.dot(hidden, wc2_ref[...],
                            preferred_element_type=jnp.float32)
                    + bc2_ref[...])


def kernel(x, edge_index, batch, graph_features, W1, b1, W2, b2, Wg, bg,
           gamma, beta, Wc1, bc1, Wc2, bc2):
    ei = edge_index.astype(jnp.int32)
    pad = jnp.full((E_PAD - E,), PADN, jnp.int32)
    src2 = jnp.concatenate([ei[0], pad]).reshape(NW, ROWS_T, CH)
    dst2 = jnp.concatenate([ei[1], pad]).reshape(NW, ROWS_T, CH)
    ones8 = jnp.ones((N_PAD, 8), jnp.float32)

    deg_kernel, scatter_kernel = _sc_kernels()
    deg = deg_kernel(dst2, ones8)

    x_pad = jnp.concatenate(
        [x, jnp.zeros((N_PAD - N, D_IN), jnp.float32)], axis=0)
    h1 = pl.pallas_call(
        _stage_a,
        out_shape=jax.ShapeDtypeStruct((N_PAD, HID), jnp.float32),
    )(x_pad, W1)
    p1, dinv = pl.pallas_call(
        _stage_b,
        out_shape=[
            jax.ShapeDtypeStruct((N_PAD, HID), jnp.bfloat16),
            jax.ShapeDtypeStruct((N_PAD, 1), jnp.float32),
        ],
    )(h1, deg)

    s1 = scatter_kernel(p1, src2, dst2)

    p2 = pl.pallas_call(
        _stage_d,
        out_shape=jax.ShapeDtypeStruct((N_PAD, HID), jnp.bfloat16),
    )(s1, p1, dinv, b1, W2)

    s2 = scatter_kernel(p2, src2, dst2)

    batch_row = jnp.concatenate(
        [batch.astype(jnp.int32),
         jnp.full((N_PAD - N,), NG, jnp.int32)]).reshape(1, N_PAD)
    out = pl.pallas_call(
        _stage_f,
        out_shape=jax.ShapeDtypeStruct((NG, 1), jnp.float32),
    )(s2, p2, dinv, b2, batch_row, graph_features, Wg, bg, gamma, beta,
      Wc1, bc1, Wc2, bc2)
    return out


# speedup vs baseline: 1.0110x; 1.0110x over previous
import functools

import jax
import jax.numpy as jnp
from jax import lax
from jax.experimental import pallas as pl
from jax.experimental.pallas import tpu as pltpu
from jax.experimental.pallas import tpu_sc as plsc

N = 10000
E = 320000
D_IN = 128
HID = 64
NG = 64
GF = 128
N_PAD = 10240
NC = 2
NS = 16
NW = NC * NS
CH = 400
ROWS_T = 25
EPT = ROWS_T * CH
E_PAD = NW * EPT
PADN = N_PAD - 1
RPT = N_PAD // NS

def _deg_body(dst_hbm, ones_hbm, out_hbm, dst_all, ones_v, bounce, acc):
    c = lax.axis_index("c")
    s = lax.axis_index("s")
    wid = s * NC + c
    pltpu.sync_copy(dst_hbm.at[wid], dst_all)
    pltpu.sync_copy(ones_hbm.at[pl.ds(0, CH)], ones_v)
    pltpu.sync_copy(ones_hbm.at[pl.ds(s * RPT, RPT)], bounce)
    pltpu.sync_copy(bounce, acc.at[pl.ds(s * RPT, RPT)])
    plsc.subcore_barrier()

    def chunk(i, carry):
        pltpu.sync_copy(ones_v, acc.at[dst_all.at[i]], add=True)
        return carry

    lax.fori_loop(0, ROWS_T, chunk, 0)
    plsc.subcore_barrier()
    pltpu.sync_copy(acc.at[pl.ds(s * RPT, RPT)], bounce)
    pltpu.sync_copy(bounce, out_hbm.at[c].at[pl.ds(s * RPT, RPT)])


def _scatter_body(p_hbm, src_hbm, dst_hbm, out_hbm,
                  src_all, dst_all, rows0, rows1, bounce, acc,
                  sem0, sem1, sem2, sem3):
    c = lax.axis_index("c")
    s = lax.axis_index("s")
    wid = s * NC + c
    pltpu.sync_copy(src_hbm.at[wid], src_all)
    pltpu.sync_copy(dst_hbm.at[wid], dst_all)
    pltpu.sync_copy(p_hbm.at[pl.ds(s * RPT, RPT)], bounce)
    pltpu.sync_copy(bounce, acc.at[pl.ds(s * RPT, RPT)])
    plsc.subcore_barrier()

    def g_start(j, buf, sem):
        pltpu.async_copy(p_hbm.at[src_all.at[j]], buf, sem)

    def g_wait(buf, sem):
        pltpu.make_async_copy(p_hbm.at[src_all.at[0]], buf, sem).wait()

    g_start(0, rows0, sem0)

    def pair(k, carry):
        i = 2 * k
        g_start(i + 1, rows1, sem1)
        g_wait(rows0, sem0)
        pltpu.sync_copy(rows0, acc.at[dst_all.at[i]], add=True)
        g_start(i + 2, rows0, sem0)
        g_wait(rows1, sem1)
        pltpu.sync_copy(rows1, acc.at[dst_all.at[i + 1]], add=True)
        return carry

    lax.fori_loop(0, (ROWS_T - 1) // 2, pair, 0)
    g_wait(rows0, sem0)
    pltpu.sync_copy(rows0, acc.at[dst_all.at[ROWS_T - 1]], add=True)
    plsc.subcore_barrier()
    pltpu.sync_copy(acc.at[pl.ds(s * RPT, RPT)], bounce)
    pltpu.sync_copy(bounce, out_hbm.at[c].at[pl.ds(s * RPT, RPT)])


@functools.cache
def _sc_kernels():
    mesh = plsc.VectorSubcoreMesh(core_axis_name="c", subcore_axis_name="s")
    deg_kernel = pl.kernel(
        _deg_body,
        mesh=mesh,
        compiler_params=pltpu.CompilerParams(use_tc_tiling_on_sc=False),
        out_type=jax.ShapeDtypeStruct((NC, N_PAD, 8), jnp.float32),
        scratch_types=[
            pltpu.VMEM((ROWS_T, CH), jnp.int32),
            pltpu.VMEM((CH, 8), jnp.float32),
            pltpu.VMEM((RPT, 8), jnp.float32),
            pltpu.VMEM_SHARED((N_PAD, 8), jnp.float32),
        ],
    )
    scatter_kernel = pl.kernel(
        _scatter_body,
        mesh=mesh,
        compiler_params=pltpu.CompilerParams(use_tc_tiling_on_sc=False),
        out_type=jax.ShapeDtypeStruct((NC, N_PAD, HID), jnp.bfloat16),
        scratch_types=[
            pltpu.VMEM((ROWS_T, CH), jnp.int32),
            pltpu.VMEM((ROWS_T, CH), jnp.int32),
            pltpu.VMEM((CH, HID), jnp.bfloat16),
            pltpu.VMEM((CH, HID), jnp.bfloat16),
            pltpu.VMEM((RPT, HID), jnp.bfloat16),
            pltpu.VMEM_SHARED((N_PAD, HID), jnp.bfloat16),
            pltpu.SemaphoreType.DMA,
            pltpu.SemaphoreType.DMA,
            pltpu.SemaphoreType.DMA,
            pltpu.SemaphoreType.DMA,
        ],
    )
    return deg_kernel, scatter_kernel


def _stage_b(x_ref, w1_ref, deg_ref, p_ref, dinv_ref):
    deg = deg_ref[0, :, 0:1] + deg_ref[1, :, 0:1] - 1.0
    dinv = lax.rsqrt(deg)
    h = jnp.dot(x_ref[...], w1_ref[...], preferred_element_type=jnp.float32)
    p_ref[...] = (h * dinv).astype(jnp.bfloat16)
    dinv_ref[...] = dinv


def _stage_d(s_ref, p_ref, dinv_ref, b1_ref, w2_ref, p2_ref):
    dinv = dinv_ref[...]
    edge_sum = (s_ref[0].astype(jnp.float32) + s_ref[1].astype(jnp.float32)
                - p_ref[...].astype(jnp.float32))
    z = jnp.maximum(edge_sum * dinv + b1_ref[...], 0.0)
    h2 = jnp.dot(z, w2_ref[...], preferred_element_type=jnp.float32)
    p2_ref[...] = (h2 * dinv).astype(jnp.bfloat16)


def _stage_f(s_ref, p_ref, dinv_ref, b2_ref, batch_ref, gf_ref, wg_ref,
             bg_ref, gamma_ref, beta_ref, wc1_ref, bc1_ref, wc2_ref,
             bc2_ref, out_ref):
    edge_sum = (s_ref[0].astype(jnp.float32) + s_ref[1].astype(jnp.float32)
                - p_ref[...].astype(jnp.float32))
    z2 = jnp.maximum(edge_sum * dinv_ref[...] + b2_ref[...], 0.0)
    seg = lax.broadcasted_iota(jnp.int32, (NG, N_PAD), 0)
    oh = (seg == batch_ref[...]).astype(jnp.float32)
    sums = jnp.dot(oh, z2, preferred_element_type=jnp.float32)
    cnt = jnp.sum(oh, axis=1, keepdims=True)
    node_feat = sums / jnp.maximum(cnt, 1.0)
    g = jnp.dot(gf_ref[...], wg_ref[...],
                preferred_element_type=jnp.float32) + bg_ref[...]
    mean = jnp.mean(g, axis=0, keepdims=True)
    var = jnp.mean((g - mean) * (g - mean), axis=0, keepdims=True)
    g = gamma_ref[...] * (g - mean) / jnp.sqrt(var + 1e-5) + beta_ref[...]
    g = jnp.maximum(g, 0.0)
    comb = jnp.concatenate([node_feat, g], axis=1)
    hidden = jnp.maximum(
        jnp.dot(comb, wc1_ref[...], preferred_element_type=jnp.float32)
        + bc1_ref[...], 0.0)
    out_ref[...] = (jnp.dot(hidden, wc2_ref[...],
                            preferred_element_type=jnp.float32)
                    + bc2_ref[...])


def kernel(x, edge_index, batch, graph_features, W1, b1, W2, b2, Wg, bg,
           gamma, beta, Wc1, bc1, Wc2, bc2):
    ei = edge_index.astype(jnp.int32)
    pad = jnp.full((E_PAD - E,), PADN, jnp.int32)
    src2 = jnp.concatenate([ei[0], pad]).reshape(NW, ROWS_T, CH)
    dst2 = jnp.concatenate([ei[1], pad]).reshape(NW, ROWS_T, CH)
    ones8 = jnp.ones((N_PAD, 8), jnp.float32)

    deg_kernel, scatter_kernel = _sc_kernels()
    deg = deg_kernel(dst2, ones8)

    x_pad = jnp.concatenate(
        [x, jnp.zeros((N_PAD - N, D_IN), jnp.float32)], axis=0)
    p1, dinv = pl.pallas_call(
        _stage_b,
        out_shape=[
            jax.ShapeDtypeStruct((N_PAD, HID), jnp.bfloat16),
            jax.ShapeDtypeStruct((N_PAD, 1), jnp.float32),
        ],
    )(x_pad, W1, deg)

    s1 = scatter_kernel(p1, src2, dst2)

    p2 = pl.pallas_call(
        _stage_d,
        out_shape=jax.ShapeDtypeStruct((N_PAD, HID), jnp.bfloat16),
    )(s1, p1, dinv, b1, W2)

    s2 = scatter_kernel(p2, src2, dst2)

    batch_row = jnp.concatenate(
        [batch.astype(jnp.int32),
         jnp.full((N_PAD - N,), NG, jnp.int32)]).reshape(1, N_PAD)
    out = pl.pallas_call(
        _stage_f,
        out_shape=jax.ShapeDtypeStruct((NG, 1), jnp.float32),
    )(s2, p2, dinv, b2, batch_row, graph_features, Wg, bg, gamma, beta,
      Wc1, bc1, Wc2, bc2)
    return out
